# baseline (speedup 1.0000x reference)
import functools

import jax
import jax.numpy as jnp
from jax import lax
from jax.experimental import pallas as pl
from jax.experimental.pallas import tpu as pltpu
from jax.experimental.pallas import tpu_sc as plsc

_B = 16384
_F = 26
_V = 100000
_D = 32
_N = _B * _F
_NC = 2
_NS = 16
_NW = _NC * _NS
_RPW = _N // _NW
_CHUNK = 1664
_NCHUNK = _RPW // _CHUNK
_G = 128
_GPC = _CHUNK // _G
_PAT = 208

_mesh = plsc.VectorSubcoreMesh(
    core_axis_name="c", subcore_axis_name="s",
    num_cores=_NC, num_subcores=_NS)


@functools.partial(
    pl.kernel,
    out_type=jax.ShapeDtypeStruct((_N, _D), jnp.float32),
    mesh=_mesh,
    scratch_types=[
        pltpu.VMEM((_CHUNK,), jnp.int32),
        pltpu.VMEM((_CHUNK, _D), jnp.float32),
        pltpu.VMEM((_PAT,), jnp.int32),
        pltpu.SemaphoreType.DMA,
    ],
    compiler_params=pltpu.CompilerParams(use_tc_tiling_on_sc=False),
)
def _gather_kernel(tab_hbm, idx_hbm, out_hbm, idx_v, rows_v, pat_v, sem):
    wid = lax.axis_index("s") * _NC + lax.axis_index("c")
    lane = lax.iota(jnp.int32, 16)
    for g in range(_PAT // 16):
        pat_v[pl.ds(g * 16, 16)] = ((lane + g * 16) % _F) * _V
    base = wid * _RPW

    def chunk_body(c, carry):
        row0 = base + c * _CHUNK
        pltpu.sync_copy(idx_hbm.at[pl.ds(row0, _CHUNK)], idx_v)
        for g in range(_CHUNK // 16):
            slot = (g % (_PAT // 16)) * 16
            idx_v[pl.ds(g * 16, 16)] = (
                idx_v[pl.ds(g * 16, 16)] + pat_v[pl.ds(slot, 16)])
        copies = []
        for j in range(_GPC):
            cp = pltpu.make_async_copy(
                tab_hbm.at[idx_v.at[pl.ds(j * _G, _G)]],
                rows_v.at[pl.ds(j * _G, _G)], sem)
            cp.start()
            copies.append(cp)
        for cp in copies:
            cp.wait()
        pltpu.sync_copy(rows_v, out_hbm.at[pl.ds(row0, _CHUNK)])
        return carry

    lax.fori_loop(0, _NCHUNK, chunk_body, None)


_VCH = 2048
_NVB = -(-_V // _VCH)


def _transpose_body(tab_ref, out_ref):
    out_ref[...] = jnp.swapaxes(tab_ref[...], 0, 1)


_tc_transpose = pl.pallas_call(
    _transpose_body,
    grid=(_F, _NVB),
    in_specs=[pl.BlockSpec((None, _D, _VCH), lambda f, c: (f, 0, c))],
    out_specs=pl.BlockSpec((None, _VCH, _D), lambda f, c: (f, c, 0)),
    out_shape=jax.ShapeDtypeStruct((_F, _V, _D), jnp.float32),
)


def kernel(indices, tables):
    idx2 = indices.reshape(_N)
    tab_vmajor = _tc_transpose(jnp.transpose(tables, (0, 2, 1)))
    tab = tab_vmajor.reshape(_F * _V, _D)
    out = _gather_kernel(tab, idx2)
    return out.reshape(_B, _F * _D)


# speedup vs baseline: 2.7473x; 2.7473x over previous
import functools

import jax
import jax.numpy as jnp
from jax import lax
from jax.experimental import pallas as pl
from jax.experimental.pallas import tpu as pltpu
from jax.experimental.pallas import tpu_sc as plsc

_B = 16384
_F = 26
_V = 100000
_D = 32
_NC = 2
_NS = 16
_NW = _NC * _NS
_ITEMS = _F * _D
_IPW = _ITEMS // _NW
_Q = 4096
_NQ = _B // _Q

_mesh = plsc.VectorSubcoreMesh(
    core_axis_name="c", subcore_axis_name="s",
    num_cores=_NC, num_subcores=_NS)


@functools.partial(
    pl.kernel,
    out_type=jax.ShapeDtypeStruct((_F * _D * _B,), jnp.float32),
    mesh=_mesh,
    scratch_types=[
        pltpu.VMEM((_V,), jnp.float32),
        pltpu.VMEM((_B,), jnp.int32),
        pltpu.VMEM((2, _Q), jnp.float32),
        pltpu.SemaphoreType.DMA,
    ],
    compiler_params=pltpu.CompilerParams(
        use_tc_tiling_on_sc=False, needs_layout_passes=False),
)
def _plane_gather_kernel(tab_hbm, idx_hbm, out_hbm, vec_v, idx_v, out_v,
                         wsem):
    wid = lax.axis_index("s") * _NC + lax.axis_index("c")
    item0 = wid * _IPW

    def out_cp(item, q, buf):
        return pltpu.make_async_copy(
            out_v.at[buf], out_hbm.at[pl.ds(item * _B + q * _Q, _Q)], wsem)

    def body(i, prev_f):
        item = item0 + i
        f = item // _D
        d = item % _D

        @pl.when(f != prev_f)
        def _():
            pltpu.sync_copy(idx_hbm.at[pl.ds(f * _B, _B)], idx_v)

        pltpu.sync_copy(tab_hbm.at[pl.ds(item * _V, _V)], vec_v)

        for q in range(_NQ):
            buf = q % 2

            def gather_body(g, carry):
                b0 = q * _Q + g * 64
                for u in range(4):
                    idx16 = idx_v[pl.ds(b0 + u * 16, 16)]
                    out_v[buf, pl.ds(g * 64 + u * 16, 16)] = (
                        plsc.load_gather(vec_v, [idx16]))
                return carry

            lax.fori_loop(0, _Q // 64, gather_body, None)
            if q >= 2:
                out_cp(item, q - 2, buf).wait()
            out_cp(item, q, buf).start()

        out_cp(item, _NQ - 2, 0).wait()
        out_cp(item, _NQ - 1, 1).wait()
        return f

    lax.fori_loop(0, _IPW, body, jnp.int32(-1))


def kernel(indices, tables):
    tab_dmaj = jnp.transpose(tables, (0, 2, 1)).reshape(_F * _D * _V)
    idx_t = jnp.transpose(indices, (1, 0)).reshape(_F * _B)
    out3 = _plane_gather_kernel(tab_dmaj, idx_t)
    return jnp.transpose(out3.reshape(_F * _D, _B), (1, 0)).reshape(
        _B, _F * _D)
